# baseline (speedup 1.0000x reference)
import functools

import jax
import jax.numpy as jnp
from jax import lax
from jax.experimental import pallas as pl
from jax.experimental.pallas import tpu as pltpu
from jax.experimental.pallas import tpu_sc as plsc

S = 64
G = 128
P = G * G
F = 6
B = 16384
GXBLK = 16
BBLK = 2048
NW = 32
BPW = B // NW
L = 16


def _table_body(sf_ref, w_ref, t_ref):
    v = w_ref[0, 0] * sf_ref[:, 0]
    for f in range(1, F):
        v = v + w_ref[0, f] * sf_ref[:, f]
    m = jnp.max(v, axis=0)
    e = jnp.exp(v - m[None])
    z = jnp.sum(e, axis=0)
    num = jnp.sum(e * v, axis=0)
    t_ref[...] = num / z


def _phi_body(p_ref, w_ref, pr_ref, idx_ref):
    def row(k, c):
        return p_ref[k, c:c + 1]

    pr_l = w_ref[0, 0] * row(0, 0)
    pr_r = w_ref[0, 0] * row(0, 1)
    for f in range(1, F):
        pr_l = pr_l + w_ref[0, f] * row(f, 0)
        pr_r = pr_r + w_ref[0, f] * row(f, 1)
    pr_ref[...] = jnp.concatenate([pr_l, pr_r], axis=0)

    def flat_idx(c):
        return row(6, c).astype(jnp.int32) * G + row(7, c).astype(jnp.int32), \
               row(8, c).astype(jnp.int32) * G + row(9, c).astype(jnp.int32)

    ss_l, es_l = flat_idx(0)
    ss_r, es_r = flat_idx(1)
    idx_ref[...] = jnp.concatenate([ss_l, ss_r, es_l, es_r], axis=0)


def _sc_combine_body(t_hbm, idx_hbm, pr_hbm, out_hbm,
                     t_v, idx_v, pr_v, out_v, sem):
    wid = lax.axis_index("s") * 2 + lax.axis_index("c")
    base = wid * BPW
    c1 = pltpu.async_copy(t_hbm, t_v, sem)
    c2 = pltpu.async_copy(idx_hbm.at[:, pl.ds(base, BPW)], idx_v, sem)
    c3 = pltpu.async_copy(pr_hbm.at[:, pl.ds(base, BPW)], pr_v, sem)
    c1.wait()
    c2.wait()
    c3.wait()

    def body(c, carry):
        s = c * L
        v_ssl = plsc.load_gather(t_v, [idx_v[0, pl.ds(s, L)]])
        v_ssr = plsc.load_gather(t_v, [idx_v[1, pl.ds(s, L)]])
        v_esl = plsc.load_gather(t_v, [idx_v[2, pl.ds(s, L)]])
        v_esr = plsc.load_gather(t_v, [idx_v[3, pl.ds(s, L)]])
        d_l = pr_v[0, pl.ds(s, L)] + v_esl - v_ssl
        d_r = pr_v[1, pl.ds(s, L)] + v_esr - v_ssr
        z = d_l - d_r
        out_v[0, pl.ds(s, L)] = 1.0 / (1.0 + jnp.exp(-z))
        out_v[1, pl.ds(s, L)] = 1.0 / (1.0 + jnp.exp(z))
        return carry

    lax.fori_loop(0, BPW // L, body, 0)
    pltpu.sync_copy(out_v, out_hbm.at[:, pl.ds(base, BPW)])


@functools.cache
def _make_sc_combine():
    return functools.partial(
        pl.kernel,
        mesh=plsc.VectorSubcoreMesh(core_axis_name="c", subcore_axis_name="s"),
        out_type=jax.ShapeDtypeStruct((2, B), jnp.float32),
        compiler_params=pltpu.CompilerParams(needs_layout_passes=False),
        scratch_types=[
            pltpu.VMEM((P,), jnp.float32),
            pltpu.VMEM((4, BPW), jnp.int32),
            pltpu.VMEM((2, BPW), jnp.float32),
            pltpu.VMEM((2, BPW), jnp.float32),
            pltpu.SemaphoreType.DMA,
        ],
    )(_sc_combine_body)


def kernel(phi, succ_feats, W):
    sfT = jnp.transpose(succ_feats, (0, 3, 1, 2))
    phiT = jnp.transpose(phi, (2, 1, 0))

    t_tab = pl.pallas_call(
        _table_body,
        grid=(G // GXBLK,),
        in_specs=[
            pl.BlockSpec((S, F, GXBLK, G), lambda j: (0, 0, j, 0)),
            pl.BlockSpec(memory_space=pltpu.SMEM),
        ],
        out_specs=pl.BlockSpec((GXBLK, G), lambda j: (j, 0)),
        out_shape=jax.ShapeDtypeStruct((G, G), jnp.float32),
    )(sfT, W)

    pr, idx = pl.pallas_call(
        _phi_body,
        grid=(B // BBLK,),
        in_specs=[
            pl.BlockSpec((10, 2, BBLK), lambda j: (0, 0, j)),
            pl.BlockSpec(memory_space=pltpu.SMEM),
        ],
        out_specs=[
            pl.BlockSpec((2, BBLK), lambda j: (0, j)),
            pl.BlockSpec((4, BBLK), lambda j: (0, j)),
        ],
        out_shape=[
            jax.ShapeDtypeStruct((2, B), jnp.float32),
            jax.ShapeDtypeStruct((4, B), jnp.int32),
        ],
    )(phiT, W)

    out = _make_sc_combine()(t_tab.reshape(P), idx, pr)
    return jnp.transpose(out, (1, 0))[:, :, None]
